# Initial kernel scaffold; baseline (speedup 1.0000x reference)
import jax
import jax.numpy as jnp
from jax.experimental import pallas as pl


def kernel(anchor, positive):
    raise NotImplementedError("write your pallas kernel here")


# speedup vs baseline: 13.1998x; 13.1998x over previous
import functools
import jax
import jax.numpy as jnp
from jax.experimental import pallas as pl


def _tc_body(a_ref, p_ref, out_ref, *, rows_per_step, batch):
    i = pl.program_id(0)
    a = a_ref[...]
    p = p_ref[...]
    a_n = a * jax.lax.rsqrt(jnp.sum(a * a, axis=1, keepdims=True))
    p_n = p * jax.lax.rsqrt(jnp.sum(p * p, axis=1, keepdims=True))
    s = jax.lax.dot_general(a_n, p_n, (((1,), (1,)), ((), ())),
                            preferred_element_type=jnp.float32)
    col = jax.lax.broadcasted_iota(jnp.int32, s.shape, 1)
    row = jax.lax.broadcasted_iota(jnp.int32, s.shape, 0) + i * rows_per_step
    diag = col == row
    d = jnp.abs(s - 1.0)
    d = jnp.where(diag | (d == 0.0), -jnp.inf, d)
    m = jnp.max(d, axis=1, keepdims=True)
    jj = jnp.where(d == m, col, jnp.int32(2**30))
    jstar = jnp.min(jj, axis=1, keepdims=True)
    an = jnp.sum(jnp.where(col == jstar, s, 0.0), axis=1)
    ap = jnp.sum(jnp.where(diag, s, 0.0), axis=1)
    partial = jnp.sum(jnp.maximum(1.0 + ap - an, 0.0)) * (1.0 / batch)

    @pl.when(i == 0)
    def _():
        out_ref[...] = jnp.zeros_like(out_ref)

    out_ref[...] += jnp.full(out_ref.shape, partial, jnp.float32)


def kernel(anchor, positive):
    batch, dim = anchor.shape
    rows_per_step = 256
    grid = batch // rows_per_step
    out = pl.pallas_call(
        functools.partial(_tc_body, rows_per_step=rows_per_step, batch=batch),
        grid=(grid,),
        in_specs=[
            pl.BlockSpec((rows_per_step, dim), lambda i: (i, 0)),
            pl.BlockSpec((batch, dim), lambda i: (0, 0)),
        ],
        out_specs=pl.BlockSpec((8, 128), lambda i: (0, 0)),
        out_shape=jax.ShapeDtypeStruct((8, 128), jnp.float32),
    )(anchor, positive)
    return out[0, 0]


# speedup vs baseline: 27.6372x; 2.0938x over previous
import functools
import jax
import jax.numpy as jnp
from jax.experimental import pallas as pl


def _tc_body(a_ref, p_ref, out_ref, *, rows_per_step, batch):
    i = pl.program_id(0)
    a = a_ref[...]
    p = p_ref[...]
    a_n = a * jax.lax.rsqrt(jnp.sum(a * a, axis=1, keepdims=True))
    p_n = p * jax.lax.rsqrt(jnp.sum(p * p, axis=1, keepdims=True))
    s = jax.lax.dot_general(a_n, p_n, (((1,), (1,)), ((), ())),
                            preferred_element_type=jnp.float32)
    col = jax.lax.broadcasted_iota(jnp.int32, s.shape, 1)
    row = jax.lax.broadcasted_iota(jnp.int32, s.shape, 0) + i * rows_per_step
    an = jnp.min(jnp.where(col == row, jnp.inf, s), axis=1)
    p_step = p_ref[pl.ds(i * rows_per_step, rows_per_step), :]
    p_n_step = p_step * jax.lax.rsqrt(
        jnp.sum(p_step * p_step, axis=1, keepdims=True))
    ap = jnp.sum(a_n * p_n_step, axis=1)
    partial = jnp.sum(jnp.maximum(1.0 + ap - an, 0.0)) * (1.0 / batch)

    @pl.when(i == 0)
    def _():
        out_ref[...] = jnp.zeros_like(out_ref)

    out_ref[...] += jnp.full(out_ref.shape, partial, jnp.float32)


def kernel(anchor, positive):
    batch, dim = anchor.shape
    rows_per_step = 256
    grid = batch // rows_per_step
    out = pl.pallas_call(
        functools.partial(_tc_body, rows_per_step=rows_per_step, batch=batch),
        grid=(grid,),
        in_specs=[
            pl.BlockSpec((rows_per_step, dim), lambda i: (i, 0)),
            pl.BlockSpec((batch, dim), lambda i: (0, 0)),
        ],
        out_specs=pl.BlockSpec((8, 128), lambda i: (0, 0)),
        out_shape=jax.ShapeDtypeStruct((8, 128), jnp.float32),
    )(anchor, positive)
    return out[0, 0]


# speedup vs baseline: 34.6173x; 1.2526x over previous
import functools
import jax
import jax.numpy as jnp
from jax.experimental import pallas as pl
from jax.experimental.pallas import tpu as pltpu


def _tc_body(a_ref, p_ref, out_ref, pn_ref, *, rows_per_step, batch):
    i = pl.program_id(0)

    @pl.when(i == 0)
    def _():
        p = p_ref[...]
        pn_ref[...] = p * jax.lax.rsqrt(jnp.sum(p * p, axis=1, keepdims=True))
        out_ref[...] = jnp.zeros_like(out_ref)

    a = a_ref[...]
    a_n = a * jax.lax.rsqrt(jnp.sum(a * a, axis=1, keepdims=True))
    p_n = pn_ref[...]
    s = jax.lax.dot_general(a_n, p_n, (((1,), (1,)), ((), ())),
                            preferred_element_type=jnp.float32)
    col = jax.lax.broadcasted_iota(jnp.int32, s.shape, 1)
    row = jax.lax.broadcasted_iota(jnp.int32, s.shape, 0) + i * rows_per_step
    an = jnp.min(jnp.where(col == row, jnp.inf, s), axis=1)
    p_n_step = pn_ref[pl.ds(i * rows_per_step, rows_per_step), :]
    ap = jnp.sum(a_n * p_n_step, axis=1)
    partial = jnp.sum(jnp.maximum(1.0 + ap - an, 0.0)) * (1.0 / batch)

    out_ref[...] += jnp.full(out_ref.shape, partial, jnp.float32)


def kernel(anchor, positive):
    batch, dim = anchor.shape
    rows_per_step = 512
    grid = batch // rows_per_step
    out = pl.pallas_call(
        functools.partial(_tc_body, rows_per_step=rows_per_step, batch=batch),
        grid=(grid,),
        in_specs=[
            pl.BlockSpec((rows_per_step, dim), lambda i: (i, 0)),
            pl.BlockSpec((batch, dim), lambda i: (0, 0)),
        ],
        out_specs=pl.BlockSpec((8, 128), lambda i: (0, 0)),
        out_shape=jax.ShapeDtypeStruct((8, 128), jnp.float32),
        scratch_shapes=[pltpu.VMEM((batch, dim), jnp.float32)],
    )(anchor, positive)
    return out[0, 0]


# speedup vs baseline: 36.4086x; 1.0517x over previous
import functools
import jax
import jax.numpy as jnp
from jax.experimental import pallas as pl
from jax.experimental.pallas import tpu as pltpu


def _tc_body(a_ref, p_ref, out_ref, pn_ref, *, rows_per_step, batch):
    i = pl.program_id(0)

    @pl.when(i == 0)
    def _():
        p = p_ref[...]
        pn_ref[...] = p * jax.lax.rsqrt(jnp.sum(p * p, axis=1, keepdims=True))
        out_ref[...] = jnp.zeros_like(out_ref)

    a = a_ref[...]
    a_n = a * jax.lax.rsqrt(jnp.sum(a * a, axis=1, keepdims=True))
    p_n = pn_ref[...]
    s = jax.lax.dot_general(a_n, p_n, (((1,), (1,)), ((), ())),
                            preferred_element_type=jnp.float32)
    col = jax.lax.broadcasted_iota(jnp.int32, s.shape, 1)
    row = jax.lax.broadcasted_iota(jnp.int32, s.shape, 0) + i * rows_per_step
    an = jnp.min(jnp.where(col == row, jnp.inf, s), axis=1)
    p_n_step = pn_ref[pl.ds(i * rows_per_step, rows_per_step), :]
    ap = jnp.sum(a_n * p_n_step, axis=1)
    partial = jnp.sum(jnp.maximum(1.0 + ap - an, 0.0)) * (1.0 / batch)

    out_ref[...] += jnp.full(out_ref.shape, partial, jnp.float32)


def kernel(anchor, positive):
    batch, dim = anchor.shape
    rows_per_step = 1024
    grid = batch // rows_per_step
    out = pl.pallas_call(
        functools.partial(_tc_body, rows_per_step=rows_per_step, batch=batch),
        grid=(grid,),
        in_specs=[
            pl.BlockSpec((rows_per_step, dim), lambda i: (i, 0)),
            pl.BlockSpec((batch, dim), lambda i: (0, 0)),
        ],
        out_specs=pl.BlockSpec((8, 128), lambda i: (0, 0)),
        out_shape=jax.ShapeDtypeStruct((8, 128), jnp.float32),
        scratch_shapes=[pltpu.VMEM((batch, dim), jnp.float32)],
    )(anchor, positive)
    return out[0, 0]


# speedup vs baseline: 42.0407x; 1.1547x over previous
import functools
import jax
import jax.numpy as jnp
from jax.experimental import pallas as pl


def _tc_body(a_ref, p_ref, out_ref, *, batch, col_chunk):
    a = a_ref[...]
    p = p_ref[...]
    a_n = a * jax.lax.rsqrt(jnp.sum(a * a, axis=1, keepdims=True))
    p_n = p * jax.lax.rsqrt(jnp.sum(p * p, axis=1, keepdims=True))

    eye = (jax.lax.broadcasted_iota(jnp.int32, (col_chunk, col_chunk), 0) ==
           jax.lax.broadcasted_iota(jnp.int32, (col_chunk, col_chunk), 1))
    chunk_mins = []
    for c in range(batch // col_chunk):
        lo = c * col_chunk
        hi = lo + col_chunk
        p_c = p_n[lo:hi, :]
        s_c = jax.lax.dot_general(a_n, p_c, (((1,), (1,)), ((), ())),
                                  preferred_element_type=jnp.float32)
        parts = []
        if lo > 0:
            parts.append(jnp.min(s_c[:lo, :], axis=1, keepdims=True))
        mid = jnp.where(eye, jnp.inf, s_c[lo:hi, :])
        parts.append(jnp.min(mid, axis=1, keepdims=True))
        if hi < batch:
            parts.append(jnp.min(s_c[hi:, :], axis=1, keepdims=True))
        chunk_mins.append(jnp.concatenate(parts, axis=0))
    an = chunk_mins[0]
    for m in chunk_mins[1:]:
        an = jnp.minimum(an, m)
    ap = jnp.sum(a_n * p_n, axis=1, keepdims=True)
    loss = jnp.sum(jnp.maximum(1.0 + ap - an, 0.0)) * (1.0 / batch)
    out_ref[...] = jnp.full(out_ref.shape, loss, jnp.float32)


def kernel(anchor, positive):
    batch, dim = anchor.shape
    out = pl.pallas_call(
        functools.partial(_tc_body, batch=batch, col_chunk=512),
        out_shape=jax.ShapeDtypeStruct((8, 128), jnp.float32),
    )(anchor, positive)
    return out[0, 0]
